# baseline (speedup 1.0000x reference)
import functools

import jax
import jax.numpy as jnp
from jax import lax
from jax.experimental import pallas as pl
from jax.experimental.pallas import tpu as pltpu
from jax.experimental.pallas import tpu_sc as plsc

NUM_WIFI = 100000
E = 128
T = 50
B = 4096
NW = 32
ROWS_PER_W = B // NW
CB = 8
NCHUNK = ROWS_PER_W // CB
NV = E // 16
OROW = T + 1


def _sc_body(rssi_hbm, bssid_hbm, table_hbm, pos_hbm, cls_hbm, out_hbm,
             idx2, r_a, r_b, obuf_a, obuf_b, posc, clsv,
             gsem_a, gsem_b, osem_a, osem_b):
    wid = lax.axis_index("s") * 2 + lax.axis_index("c")
    wbase = wid * ROWS_PER_W

    pltpu.sync_copy(bssid_hbm.at[pl.ds(wbase, ROWS_PER_W), :], idx2)

    pltpu.sync_copy(pos_hbm.at[pl.ds(0, OROW * E)], posc)
    pltpu.sync_copy(cls_hbm, clsv)
    for j in range(NV):
        s = pl.ds(j * 16, 16)
        posc[s] = posc[s] + clsv[s]
    for obuf in (obuf_a, obuf_b):
        for bi in range(CB):
            for j in range(NV):
                obuf[bi * OROW, pl.ds(j * 16, 16)] = posc[pl.ds(j * 16, 16)]

    def out_region(c):
        return out_hbm.at[pl.ds((wbase + c * CB) * OROW, CB * OROW), :]

    def prefetch(c, obuf, rbuf, gsem):
        for bi in range(CB):
            pltpu.async_copy(
                table_hbm.at[idx2.at[c * CB + bi]],
                obuf.at[pl.ds(bi * OROW + 1, T), :], gsem)
        pltpu.async_copy(rssi_hbm.at[pl.ds((wbase + c * CB) * T, CB * T)],
                         rbuf.at[pl.ds(0, CB * T)], gsem)

    def compute(c, obuf, rbuf, gsem):
        for bi in range(CB):
            pltpu.make_async_copy(
                table_hbm.at[idx2.at[c * CB + bi]],
                obuf.at[pl.ds(bi * OROW + 1, T), :], gsem).wait()
        pltpu.make_async_copy(rssi_hbm.at[pl.ds(0, CB * T)],
                              rbuf.at[pl.ds(0, CB * T)], gsem).wait()

        def t_body(t, carry):
            pcs = [posc[pl.ds((t + 1) * E + j * 16, 16)] for j in range(NV)]
            for bi in range(CB):
                rv = rbuf[pl.ds(bi * T + t, 16)]
                bc = jnp.full((16,), rv[0], dtype=jnp.float32)
                orow = bi * OROW + 1 + t
                for j in range(NV):
                    s = pl.ds(j * 16, 16)
                    obuf[orow, s] = obuf[orow, s] + (pcs[j] + bc)
            return carry

        lax.fori_loop(0, T, t_body, 0)

    prefetch(0, obuf_a, r_a, gsem_a)
    pltpu.async_copy(obuf_a, out_region(0), osem_a)
    pltpu.async_copy(obuf_b, out_region(1), osem_b)
    prefetch(1, obuf_b, r_b, gsem_b)

    def chunk_pair(c2, carry):
        c = 2 * c2
        compute(c, obuf_a, r_a, gsem_a)
        pltpu.async_copy(obuf_a, out_region(c), osem_a)
        compute(c + 1, obuf_b, r_b, gsem_b)
        pltpu.async_copy(obuf_b, out_region(c + 1), osem_b)

        @pl.when(c2 < NCHUNK // 2 - 1)
        def _():
            pltpu.make_async_copy(obuf_a, out_region(c), osem_a).wait()
            prefetch(c + 2, obuf_a, r_a, gsem_a)
            pltpu.make_async_copy(obuf_b, out_region(c + 1), osem_b).wait()
            prefetch(c + 3, obuf_b, r_b, gsem_b)

        return carry

    lax.fori_loop(0, NCHUNK // 2, chunk_pair, 0)

    pltpu.make_async_copy(obuf_a, out_region(NCHUNK - 2), osem_a).wait()
    pltpu.make_async_copy(obuf_b, out_region(NCHUNK - 1), osem_b).wait()


@jax.jit
def _anchor2token(rssi_f, bssid2, table, pos_f, cls_f):
    mesh = plsc.VectorSubcoreMesh(core_axis_name="c", subcore_axis_name="s")
    k = functools.partial(
        pl.kernel,
        mesh=mesh,
        out_type=jax.ShapeDtypeStruct((B * OROW, E), jnp.float32),
        scratch_types=[
            pltpu.VMEM((ROWS_PER_W, T), jnp.int32),
            pltpu.VMEM((CB * T + 16,), jnp.float32),
            pltpu.VMEM((CB * T + 16,), jnp.float32),
            pltpu.VMEM((CB * OROW, E), jnp.float32),
            pltpu.VMEM((CB * OROW, E), jnp.float32),
            pltpu.VMEM((OROW * E,), jnp.float32),
            pltpu.VMEM((E,), jnp.float32),
            pltpu.SemaphoreType.DMA,
            pltpu.SemaphoreType.DMA,
            pltpu.SemaphoreType.DMA,
            pltpu.SemaphoreType.DMA,
        ],
    )(_sc_body)
    return k(rssi_f, bssid2, table, pos_f, cls_f)


def kernel(rssi, bssid, bssid_table, pos_table, cls_token):
    rssi_f = rssi.reshape(B * T)
    bssid2 = bssid.astype(jnp.int32)
    pos_f = pos_table.reshape(pos_table.shape[0] * E)
    cls_f = cls_token.reshape(E)
    out = _anchor2token(rssi_f, bssid2, bssid_table, pos_f, cls_f)
    return out.reshape(B, T + 1, E)


# speedup vs baseline: 1.1138x; 1.1138x over previous
import functools

import jax
import jax.numpy as jnp
from jax import lax
from jax.experimental import pallas as pl
from jax.experimental.pallas import tpu as pltpu
from jax.experimental.pallas import tpu_sc as plsc

NUM_WIFI = 100000
E = 128
T = 50
B = 4096
NW = 32
ROWS_PER_W = B // NW
CB = 4
NCHUNK = ROWS_PER_W // CB
NV = E // 16
OROW = T + 1
OWORDS = CB * OROW * E


def _sc_body(rssi_hbm, bssid_hbm, table_hbm, pos_hbm, cls_hbm, out_hbm,
             idx_all, rssi_all, gbuf_a, gbuf_b, obuf_a, obuf_b,
             posc, clsv, gsem_a, gsem_b, osem_a, osem_b):
    wid = lax.axis_index("s") * 2 + lax.axis_index("c")
    wbase = wid * ROWS_PER_W

    pltpu.sync_copy(bssid_hbm.at[pl.ds(wbase * T, ROWS_PER_W * T)], idx_all)
    pltpu.sync_copy(rssi_hbm.at[pl.ds(wbase * T, ROWS_PER_W * T)],
                    rssi_all.at[pl.ds(0, ROWS_PER_W * T)])

    pltpu.sync_copy(pos_hbm.at[pl.ds(0, 56), :], posc)
    pltpu.sync_copy(cls_hbm, clsv)
    for j in range(NV):
        s = pl.ds(j * 16, 16)
        posc[0, s] = posc[0, s] + clsv[s]
    for obuf in (obuf_a, obuf_b):
        for bi in range(CB):
            for j in range(NV):
                obuf[pl.ds(bi * OROW * E + j * 16, 16)] = posc[0, pl.ds(j * 16, 16)]

    def out_region(c):
        return out_hbm.at[pl.ds((wbase + c * CB) * OROW * E, OWORDS)]

    def idx_slice(c):
        return idx_all.at[pl.ds(c * CB * T, CB * T)]

    def prefetch(c, gbuf, gsem):
        pltpu.async_copy(table_hbm.at[idx_slice(c)], gbuf, gsem)

    HW = CB // 2 * OROW * E

    def half_region(c, h):
        return out_hbm.at[pl.ds((wbase + c * CB) * OROW * E + h * HW, HW)]

    def compute(c, gbuf, gsem, obuf, osem):
        pltpu.make_async_copy(table_hbm.at[idx_slice(c)], gbuf, gsem).wait()
        for h in range(2):
            pltpu.make_async_copy(
                obuf.at[pl.ds(h * HW, HW)], half_region(c, h), osem).wait()

        for h in range(2):
            def t_body(t, carry):
                pcs = [posc[t + 1, pl.ds(j * 16, 16)] for j in range(NV)]
                for bi in (2 * h, 2 * h + 1):
                    rv = rssi_all[pl.ds(c * CB * T + bi * T + t, 16)]
                    bc = jnp.full((16,), rv[0], dtype=jnp.float32)
                    rg = bi * T + t
                    ob = (bi * OROW + 1 + t) * E
                    for j in range(NV):
                        obuf[pl.ds(ob + j * 16, 16)] = (
                            gbuf[rg, pl.ds(j * 16, 16)] + (pcs[j] + bc))
                return carry

            lax.fori_loop(0, T, t_body, 0)
            pltpu.async_copy(obuf.at[pl.ds(h * HW, HW)], half_region(c, h), osem)

    prefetch(0, gbuf_a, gsem_a)
    for h in range(2):
        pltpu.async_copy(obuf_a.at[pl.ds(h * HW, HW)], half_region(0, h), osem_a)
        pltpu.async_copy(obuf_b.at[pl.ds(h * HW, HW)], half_region(1, h), osem_b)

    def chunk_pair(c2, carry):
        c = 2 * c2
        prefetch(c + 1, gbuf_b, gsem_b)
        compute(c, gbuf_a, gsem_a, obuf_a, osem_a)

        @pl.when(c2 < NCHUNK // 2 - 1)
        def _():
            prefetch(c + 2, gbuf_a, gsem_a)

        compute(c + 1, gbuf_b, gsem_b, obuf_b, osem_b)
        return carry

    lax.fori_loop(0, NCHUNK // 2, chunk_pair, 0)

    for h in range(2):
        pltpu.make_async_copy(obuf_a.at[pl.ds(h * HW, HW)],
                              half_region(NCHUNK - 2, h), osem_a).wait()
        pltpu.make_async_copy(obuf_b.at[pl.ds(h * HW, HW)],
                              half_region(NCHUNK - 1, h), osem_b).wait()


@jax.jit
def _anchor2token(rssi_f, bssid_f, table, pos, cls_f):
    mesh = plsc.VectorSubcoreMesh(core_axis_name="c", subcore_axis_name="s")
    k = functools.partial(
        pl.kernel,
        mesh=mesh,
        out_type=jax.ShapeDtypeStruct((B * OROW * E,), jnp.float32),
        scratch_types=[
            pltpu.VMEM((ROWS_PER_W * T,), jnp.int32),
            pltpu.VMEM((ROWS_PER_W * T + 16,), jnp.float32),
            pltpu.VMEM((CB * T, E), jnp.float32),
            pltpu.VMEM((CB * T, E), jnp.float32),
            pltpu.VMEM((OWORDS,), jnp.float32),
            pltpu.VMEM((OWORDS,), jnp.float32),
            pltpu.VMEM((56, E), jnp.float32),
            pltpu.VMEM((E,), jnp.float32),
            pltpu.SemaphoreType.DMA,
            pltpu.SemaphoreType.DMA,
            pltpu.SemaphoreType.DMA,
            pltpu.SemaphoreType.DMA,
        ],
    )(_sc_body)
    return k(rssi_f, bssid_f, table, pos, cls_f)


def kernel(rssi, bssid, bssid_table, pos_table, cls_token):
    rssi_f = rssi.reshape(B * T)
    bssid_f = bssid.reshape(B * T).astype(jnp.int32)
    cls_f = cls_token.reshape(E)
    out = _anchor2token(rssi_f, bssid_f, bssid_table, pos_table, cls_f)
    return out.reshape(B, T + 1, E)
